# baseline (device time: 174078 ns/iter reference)
import os

import jax
import jax.numpy as jnp
from jax import lax
from jax.experimental import pallas as pl
from jax.experimental.pallas import tpu as pltpu

_ABLATE = set(os.environ.get("ABLATE", "").split(","))

N_DEV = 4
M, N = 4096, 2048
CM = M // N_DEV
HALF = N // 2
SUB = 2
SM = CM // SUB
N_HOPS = 2 * (N_DEV - 1)


def kernel(x, w_mat):
    x = x.astype(jnp.bfloat16)
    w = w_mat.astype(jnp.bfloat16)

    def body(x_ref, w_ref, out_ref,
             commR, commL, sbufR, sbufL, accR, accL,
             my_scal, scal_ref,
             send_semsR, recv_semsR, send_semsL, recv_semsL,
             copy_semsR, copy_semsL,
             scal_send_sems, scal_recv_sems,
             creditR, creditL):
        me = lax.axis_index("i")
        right = lax.rem(me + 1, N_DEV)
        left = lax.rem(me + N_DEV - 1, N_DEV)
        diag = lax.rem(me + 2, N_DEV)

        barrier_sem = pltpu.get_barrier_semaphore()
        for nbr in (left, right):
            pl.semaphore_signal(
                barrier_sem, inc=1,
                device_id=(nbr,), device_id_type=pl.DeviceIdType.MESH,
            )
        pl.semaphore_wait(barrier_sem, 2)

        R = dict(comm=commR, sbuf=sbufR, acc=accR, ssem=send_semsR,
                 rsem=recv_semsR, csem=copy_semsR, credit=creditR,
                 tgt=right, credit_to=left, col=0)
        L = dict(comm=commL, sbuf=sbufL, acc=accL, ssem=send_semsL,
                 rsem=recv_semsL, csem=copy_semsL, credit=creditL,
                 tgt=left, credit_to=right, col=HALF)
        DIRS = (R, L)

        def subrows(k):
            return pl.ds(k * SM, SM)

        def xrows(c, k):
            return pl.ds(c * CM + k * SM, SM)

        def colslice(d):
            return pl.ds(d['col'], HALF)

        def chunk_R(h):
            return lax.rem(me + (N_DEV - 1 - h), N_DEV)

        def chunk_L(h):
            return lax.rem(me + h + 1, N_DEV)

        def dot_sub(d, c, k):
            d['acc'][subrows(k), :] = jnp.dot(
                x_ref[xrows(c, k), :], w_ref[:, colslice(d)],
                preferred_element_type=jnp.float32,
            )

        def rdma_send(d, hop, k, src_ref):
            rd = pltpu.make_async_remote_copy(
                src_ref=src_ref,
                dst_ref=d['comm'].at[(hop % 2) * SUB + k],
                send_sem=d['ssem'].at[hop * SUB + k],
                recv_sem=d['rsem'].at[hop * SUB + k],
                device_id=(d['tgt'],),
                device_id_type=pl.DeviceIdType.MESH,
            )
            rd.start()
            return rd

        def rdma_recv_wait(d, hop, k):
            slot = (hop % 2) * SUB + k
            pltpu.make_async_remote_copy(
                src_ref=d['comm'].at[slot],
                dst_ref=d['comm'].at[slot],
                send_sem=d['ssem'].at[hop * SUB + k],
                recv_sem=d['rsem'].at[hop * SUB + k],
                device_id=(d['tgt'],),
                device_id_type=pl.DeviceIdType.MESH,
            ).wait_recv()

        def take_credit(d):
            pl.semaphore_wait(d['credit'], 1)

        def give_credit(d):
            pl.semaphore_signal(
                d['credit'], inc=1,
                device_id=(d['credit_to'],),
                device_id_type=pl.DeviceIdType.MESH,
            )

        sends = {id(d): {h: [] for h in range(N_HOPS)} for d in DIRS}

        for k in range(SUB):
            for d in DIRS:
                if "nodot" in _ABLATE:
                    d['sbuf'][k, :, :] = x_ref[xrows(me, k), :]
                else:
                    d['sbuf'][k, :, :] = jnp.dot(
                        x_ref[xrows(me, k), :], w_ref[:, colslice(d)],
                        preferred_element_type=jnp.float32,
                    ).astype(jnp.bfloat16)
                sends[id(d)][0].append(rdma_send(d, 0, k, d['sbuf'].at[k]))
        def dot_chunk(d, c):
            if "nodot" in _ABLATE:
                return
            d['acc'][...] = jnp.dot(
                x_ref[pl.ds(c * CM, CM), :], w_ref[:, colslice(d)],
                preferred_element_type=jnp.float32,
            )

        dot_chunk(R, chunk_R(0))
        dot_chunk(L, chunk_L(0))

        for h in range(N_DEV - 1):
            if h >= 1:
                for d in DIRS:
                    take_credit(d)
            if h >= 1:
                for d in DIRS:
                    for rd in sends[id(d)][h - 1]:
                        rd.wait_send()
            nxt = ((h + 1) % 2) * SUB
            for k in range(SUB):
                for d in DIRS:
                    rdma_recv_wait(d, h, k)
                    slot = (h % 2) * SUB + k
                    if "noadd" in _ABLATE:
                        d['sbuf'][nxt + k, :, :] = d['comm'][slot, :, :]
                    else:
                        d['sbuf'][nxt + k, :, :] = (
                            d['acc'][subrows(k), :]
                            + d['comm'][slot, :, :].astype(jnp.float32)
                        ).astype(jnp.bfloat16)
                    sends[id(d)][h + 1].append(
                        rdma_send(d, h + 1, k, d['sbuf'].at[nxt + k])
                    )
            for d in DIRS:
                give_credit(d)
            if h < N_DEV - 2:
                dot_chunk(R, chunk_R(h + 1))
                dot_chunk(L, chunk_L(h + 1))

        m = jnp.float32(0.0)
        for d in DIRS:
            for k in range(SUB):
                m = jnp.maximum(
                    m,
                    jnp.max(jnp.abs(d['sbuf'][SUB + k, :, :]).astype(
                        jnp.float32)),
                )
        my_scal[...] = jnp.full((8, 128), m, jnp.float32)
        scal_rdmas = []
        for j, tgt in enumerate((right, left, diag)):
            rd = pltpu.make_async_remote_copy(
                src_ref=my_scal,
                dst_ref=scal_ref.at[j],
                send_sem=scal_send_sems.at[j],
                recv_sem=scal_recv_sems.at[j],
                device_id=(tgt,),
                device_id_type=pl.DeviceIdType.MESH,
            )
            rd.start()
            scal_rdmas.append(rd)
        for rd in scal_rdmas:
            rd.wait()
        amax = jnp.maximum(m, jnp.max(scal_ref[...]))
        scale = amax / 448.0
        inv = 1.0 / scale

        pend = {id(d): [None] * (2 * SUB) for d in DIRS}

        def quant_sub(d, src_slot, qflat, dst_c, k):
            if pend[id(d)][qflat] is not None:
                pend[id(d)][qflat].wait()
            if "noquant" in _ABLATE:
                d['sbuf'][qflat, :, :] = src_slot[:, :]
            else:
                blk = src_slot[:, :].astype(jnp.float32) * inv
                q = blk.astype(jnp.float8_e4m3fn).astype(jnp.float32) * scale
                d['sbuf'][qflat, :, :] = q.astype(jnp.bfloat16)
            cp = pltpu.make_async_copy(
                d['sbuf'].at[qflat],
                out_ref.at[pl.ds(dst_c * CM + k * SM, SM), colslice(d)],
                d['csem'].at[qflat],
            )
            cp.start()
            pend[id(d)][qflat] = cp

        for d in DIRS:
            for rd in sends[id(d)][2]:
                rd.wait_send()
        ownR = lax.rem(me + 1, N_DEV)
        ownL = lax.rem(me + N_DEV - 1, N_DEV)
        for k in range(SUB):
            quant_sub(R, R['sbuf'][SUB + k], k, ownR, k)
            quant_sub(L, L['sbuf'][SUB + k], k, ownL, k)

        for d in DIRS:
            take_credit(d)
            for rd in sends[id(d)][3]:
                rd.wait_send()
        for k in range(SUB):
            for d in DIRS:
                rdma_recv_wait(d, 3, k)
                slot = SUB + k
                sends[id(d)][4].append(
                    rdma_send(d, 4, k, d['comm'].at[slot])
                )
                quant_sub(d, d['comm'][slot], SUB + k, me, k)
        for d in DIRS:
            for rd in sends[id(d)][4]:
                rd.wait_send()
            give_credit(d)

        dcR4 = lax.rem(me + N_DEV - 1, N_DEV)
        dcL4 = lax.rem(me + 1, N_DEV)
        for d in DIRS:
            take_credit(d)
        for k in range(SUB):
            for d in DIRS:
                rdma_recv_wait(d, 4, k)
                slot = k
                sends[id(d)][5].append(
                    rdma_send(d, 5, k, d['comm'].at[slot])
                )
                quant_sub(d, d['comm'][slot], k,
                          dcR4 if d is R else dcL4, k)
        for d in DIRS:
            for rd in sends[id(d)][5]:
                rd.wait_send()

        dc5 = lax.rem(me + 2, N_DEV)
        for k in range(SUB):
            for d in DIRS:
                rdma_recv_wait(d, 5, k)
                quant_sub(d, d['comm'][SUB + k], SUB + k, dc5, k)

        for d in DIRS:
            for cp in pend[id(d)]:
                if cp is not None:
                    cp.wait()

    return pl.pallas_call(
        body,
        out_shape=jax.ShapeDtypeStruct((M, N), jnp.bfloat16),
        in_specs=[
            pl.BlockSpec(memory_space=pltpu.MemorySpace.VMEM),
            pl.BlockSpec(memory_space=pltpu.MemorySpace.VMEM),
        ],
        out_specs=pl.BlockSpec(memory_space=pl.ANY),
        scratch_shapes=[
            pltpu.VMEM((2 * SUB, SM, HALF), jnp.bfloat16),
            pltpu.VMEM((2 * SUB, SM, HALF), jnp.bfloat16),
            pltpu.VMEM((2 * SUB, SM, HALF), jnp.bfloat16),
            pltpu.VMEM((2 * SUB, SM, HALF), jnp.bfloat16),
            pltpu.VMEM((CM, HALF), jnp.float32),
            pltpu.VMEM((CM, HALF), jnp.float32),
            pltpu.VMEM((8, 128), jnp.float32),
            pltpu.VMEM((3, 8, 128), jnp.float32),
            pltpu.SemaphoreType.DMA((N_HOPS * SUB,)),
            pltpu.SemaphoreType.DMA((N_HOPS * SUB,)),
            pltpu.SemaphoreType.DMA((N_HOPS * SUB,)),
            pltpu.SemaphoreType.DMA((N_HOPS * SUB,)),
            pltpu.SemaphoreType.DMA((2 * SUB,)),
            pltpu.SemaphoreType.DMA((2 * SUB,)),
            pltpu.SemaphoreType.DMA((3,)),
            pltpu.SemaphoreType.DMA((3,)),
            pltpu.SemaphoreType.REGULAR,
            pltpu.SemaphoreType.REGULAR,
        ],
        compiler_params=pltpu.CompilerParams(collective_id=0),
    )(x, w)


# device time: 140046 ns/iter; 1.2430x vs baseline; 1.2430x over previous
import os

import jax
import jax.numpy as jnp
from jax import lax
from jax.experimental import pallas as pl
from jax.experimental.pallas import tpu as pltpu

_ABLATE = set(os.environ.get("ABLATE", "").split(","))

N_DEV = 4
M, N = 4096, 2048
CM = M // N_DEV
HALF = N // 2
SUB = 4
SM = CM // SUB
N_HOPS = 2 * (N_DEV - 1)


def kernel(x, w_mat):
    x = x.astype(jnp.bfloat16)
    w = w_mat.astype(jnp.bfloat16)

    def body(x_ref, w_ref, out_ref,
             commR, commL, commqR, commqL, sbufR, sbufL, accR, accL,
             my_scal, scal_ref,
             send_semsR, recv_semsR, send_semsL, recv_semsL,
             copy_semsR, copy_semsL,
             scal_send_sems, scal_recv_sems,
             creditR, creditL):
        me = lax.axis_index("i")
        right = lax.rem(me + 1, N_DEV)
        left = lax.rem(me + N_DEV - 1, N_DEV)
        diag = lax.rem(me + 2, N_DEV)

        barrier_sem = pltpu.get_barrier_semaphore()
        for nbr in (left, right):
            pl.semaphore_signal(
                barrier_sem, inc=1,
                device_id=(nbr,), device_id_type=pl.DeviceIdType.MESH,
            )
        pl.semaphore_wait(barrier_sem, 2)

        R = dict(comm=commR, commq=commqR, sbuf=sbufR, acc=accR,
                 ssem=send_semsR, rsem=recv_semsR, csem=copy_semsR,
                 credit=creditR, tgt=right, credit_to=left, col=0)
        L = dict(comm=commL, commq=commqL, sbuf=sbufL, acc=accL,
                 ssem=send_semsL, rsem=recv_semsL, csem=copy_semsL,
                 credit=creditL, tgt=left, credit_to=right, col=HALF)
        DIRS = (R, L)

        def subrows(k):
            return pl.ds(k * SM, SM)

        def xrows(c, k):
            return pl.ds(c * CM + k * SM, SM)

        def colslice(d):
            return pl.ds(d['col'], HALF)

        def chunk_R(h):
            return lax.rem(me + (N_DEV - 1 - h), N_DEV)

        def chunk_L(h):
            return lax.rem(me + h + 1, N_DEV)

        def dot_sub(d, c, k):
            if "nodot" in _ABLATE:
                return
            d['acc'][subrows(k), :] = jnp.dot(
                x_ref[xrows(c, k), :], w_ref[:, colslice(d)],
                preferred_element_type=jnp.float32,
            )

        def rdma_send(d, hop, k, src_ref, dst_buf, dst_flat):
            rd = pltpu.make_async_remote_copy(
                src_ref=src_ref,
                dst_ref=d[dst_buf].at[dst_flat],
                send_sem=d['ssem'].at[hop * SUB + k],
                recv_sem=d['rsem'].at[hop * SUB + k],
                device_id=(d['tgt'],),
                device_id_type=pl.DeviceIdType.MESH,
            )
            rd.start()
            return rd

        def rdma_recv_wait(d, hop, k, buf, flat):
            pltpu.make_async_remote_copy(
                src_ref=d[buf].at[flat],
                dst_ref=d[buf].at[flat],
                send_sem=d['ssem'].at[hop * SUB + k],
                recv_sem=d['rsem'].at[hop * SUB + k],
                device_id=(d['tgt'],),
                device_id_type=pl.DeviceIdType.MESH,
            ).wait_recv()

        def take_credit(d):
            pl.semaphore_wait(d['credit'], 1)

        def give_credit(d):
            pl.semaphore_signal(
                d['credit'], inc=1,
                device_id=(d['credit_to'],),
                device_id_type=pl.DeviceIdType.MESH,
            )

        sends = {id(d): {h: [] for h in range(N_HOPS)} for d in DIRS}

        for k in range(SUB):
            for d in DIRS:
                if "nodot" in _ABLATE:
                    d['sbuf'][k, :, :] = x_ref[xrows(me, k), :]
                else:
                    d['sbuf'][k, :, :] = jnp.dot(
                        x_ref[xrows(me, k), :], w_ref[:, colslice(d)],
                        preferred_element_type=jnp.float32,
                    ).astype(jnp.bfloat16)
                sends[id(d)][0].append(
                    rdma_send(d, 0, k, d['sbuf'].at[k], 'comm', k))
        for k in range(SUB):
            dot_sub(R, chunk_R(0), k)
            dot_sub(L, chunk_L(0), k)

        m = jnp.float32(0.0)
        for h in range(N_DEV - 1):
            if h == 1:
                for d in DIRS:
                    take_credit(d)
            if h >= 1:
                for d in DIRS:
                    for rd in sends[id(d)][h - 1]:
                        rd.wait_send()
            nxt = ((h + 1) % 2) * SUB
            for k in range(SUB):
                for d in DIRS:
                    rdma_recv_wait(d, h, k, 'comm', (h % 2) * SUB + k)
                    slot = (h % 2) * SUB + k
                    if "noadd" in _ABLATE:
                        d['sbuf'][nxt + k, :, :] = d['comm'][slot, :, :]
                    else:
                        d['sbuf'][nxt + k, :, :] = (
                            d['acc'][subrows(k), :]
                            + d['comm'][slot, :, :].astype(jnp.float32)
                        ).astype(jnp.bfloat16)
                    if h < N_DEV - 2:
                        sends[id(d)][h + 1].append(
                            rdma_send(d, h + 1, k, d['sbuf'].at[nxt + k],
                                      'comm', nxt + k))
                        dot_sub(d, chunk_R(h + 1) if d is R else chunk_L(h + 1), k)
                    else:
                        m = jnp.maximum(
                            m,
                            jnp.max(jnp.abs(
                                d['sbuf'][nxt + k, :, :]).astype(jnp.float32)),
                        )
            if h == 0:
                for d in DIRS:
                    give_credit(d)

        my_scal[...] = jnp.full((8, 128), m, jnp.float32)
        scal_rdmas = []
        for j, tgt in enumerate((right, left, diag)):
            rd = pltpu.make_async_remote_copy(
                src_ref=my_scal,
                dst_ref=scal_ref.at[j],
                send_sem=scal_send_sems.at[j],
                recv_sem=scal_recv_sems.at[j],
                device_id=(tgt,),
                device_id_type=pl.DeviceIdType.MESH,
            )
            rd.start()
            scal_rdmas.append(rd)
        for rd in scal_rdmas:
            rd.wait()
        amax = jnp.maximum(m, jnp.max(scal_ref[...]))
        scale = amax / 448.0
        inv = 1.0 / scale

        pend = {id(d): [None] * (2 * SUB) for d in DIRS}

        def dequant_store(d, qflat, cflat, dst_c, k):
            if pend[id(d)][cflat] is not None:
                pend[id(d)][cflat].wait()
            d['comm'][cflat, :, :] = (
                d['commq'][qflat, :, :].astype(jnp.float32) * scale
            ).astype(jnp.bfloat16)
            cp = pltpu.make_async_copy(
                d['comm'].at[cflat],
                out_ref.at[pl.ds(dst_c * CM + k * SM, SM), colslice(d)],
                d['csem'].at[cflat],
            )
            cp.start()
            pend[id(d)][cflat] = cp

        ownR = lax.rem(me + 1, N_DEV)
        ownL = lax.rem(me + N_DEV - 1, N_DEV)
        for k in range(SUB):
            for d in DIRS:
                d['commq'][2 * SUB + k, :, :] = (
                    d['sbuf'][SUB + k, :, :].astype(jnp.float32) * inv
                ).astype(jnp.float8_e4m3fn)
                sends[id(d)][3].append(
                    rdma_send(d, 3, k, d['commq'].at[2 * SUB + k],
                              'commq', k))
        for d in DIRS:
            for rd in sends[id(d)][2]:
                rd.wait_send()
        for k in range(SUB):
            dequant_store(R, 2 * SUB + k, k, ownR, k)
            dequant_store(L, 2 * SUB + k, k, ownL, k)

        for k in range(SUB):
            for d in DIRS:
                rdma_recv_wait(d, 3, k, 'commq', k)
                sends[id(d)][4].append(
                    rdma_send(d, 4, k, d['commq'].at[k], 'commq', SUB + k))
                dequant_store(d, k, SUB + k, me, k)
        for d in DIRS:
            for rd in sends[id(d)][3]:
                rd.wait_send()
            for rd in sends[id(d)][4]:
                rd.wait_send()
            give_credit(d)

        dcR4 = lax.rem(me + N_DEV - 1, N_DEV)
        dcL4 = lax.rem(me + 1, N_DEV)
        for d in DIRS:
            take_credit(d)
        for k in range(SUB):
            for d in DIRS:
                rdma_recv_wait(d, 4, k, 'commq', SUB + k)
                sends[id(d)][5].append(
                    rdma_send(d, 5, k, d['commq'].at[SUB + k], 'commq', k))
                dequant_store(d, SUB + k, k, dcR4 if d is R else dcL4, k)
        for d in DIRS:
            for rd in sends[id(d)][5]:
                rd.wait_send()

        dc5 = lax.rem(me + 2, N_DEV)
        for k in range(SUB):
            for d in DIRS:
                rdma_recv_wait(d, 5, k, 'commq', k)
                dequant_store(d, k, SUB + k, dc5, k)

        for d in DIRS:
            for cp in pend[id(d)]:
                if cp is not None:
                    cp.wait()

    return pl.pallas_call(
        body,
        out_shape=jax.ShapeDtypeStruct((M, N), jnp.bfloat16),
        in_specs=[
            pl.BlockSpec(memory_space=pltpu.MemorySpace.VMEM),
            pl.BlockSpec(memory_space=pltpu.MemorySpace.VMEM),
        ],
        out_specs=pl.BlockSpec(memory_space=pl.ANY),
        scratch_shapes=[
            pltpu.VMEM((2 * SUB, SM, HALF), jnp.bfloat16),
            pltpu.VMEM((2 * SUB, SM, HALF), jnp.bfloat16),
            pltpu.VMEM((3 * SUB, SM, HALF), jnp.float8_e4m3fn),
            pltpu.VMEM((3 * SUB, SM, HALF), jnp.float8_e4m3fn),
            pltpu.VMEM((2 * SUB, SM, HALF), jnp.bfloat16),
            pltpu.VMEM((2 * SUB, SM, HALF), jnp.bfloat16),
            pltpu.VMEM((CM, HALF), jnp.float32),
            pltpu.VMEM((CM, HALF), jnp.float32),
            pltpu.VMEM((8, 128), jnp.float32),
            pltpu.VMEM((3, 8, 128), jnp.float32),
            pltpu.SemaphoreType.DMA((N_HOPS * SUB,)),
            pltpu.SemaphoreType.DMA((N_HOPS * SUB,)),
            pltpu.SemaphoreType.DMA((N_HOPS * SUB,)),
            pltpu.SemaphoreType.DMA((N_HOPS * SUB,)),
            pltpu.SemaphoreType.DMA((2 * SUB,)),
            pltpu.SemaphoreType.DMA((2 * SUB,)),
            pltpu.SemaphoreType.DMA((3,)),
            pltpu.SemaphoreType.DMA((3,)),
            pltpu.SemaphoreType.REGULAR,
            pltpu.SemaphoreType.REGULAR,
        ],
        compiler_params=pltpu.CompilerParams(collective_id=0),
    )(x, w)
